# Initial kernel scaffold; baseline (speedup 1.0000x reference)
import jax
import jax.numpy as jnp
from jax.experimental import pallas as pl


def kernel(x, v, neighbors_indices, theta1, theta2, theta3):
    raise NotImplementedError("write your pallas kernel here")


# speedup vs baseline: 67.9115x; 67.9115x over previous
import functools

import jax
import jax.numpy as jnp
from jax import lax
from jax.experimental import pallas as pl
from jax.experimental.pallas import tpu as pltpu
from jax.experimental.pallas import tpu_sc as plsc

N = 50000
NBR = 16

NC = 2
NS = 16
NW = NC * NS

BLK = 112
NBLK = 14
NPT = BLK * NBLK
NP = NPT * NW

RB = 1792


def _kron_body(x_ref, v_ref, k_ref, s_ref):
    i = pl.program_id(0)
    x = x_ref[...]
    v = v_ref[...]
    k0 = v * x[:, 0:1]
    k1 = v * x[:, 1:2]
    k = jnp.concatenate([k0, k1], axis=1)
    k_ref[...] = k

    @pl.when(i == 0)
    def _():
        s_ref[...] = jnp.zeros_like(s_ref)

    s_ref[...] += jnp.sum(k, axis=0, keepdims=True)


_kron_call = pl.pallas_call(
    _kron_body,
    grid=(NP // RB,),
    in_specs=[
        pl.BlockSpec((RB, 2), lambda i: (i, 0)),
        pl.BlockSpec((RB, 32), lambda i: (i, 0)),
    ],
    out_specs=[
        pl.BlockSpec((RB, 64), lambda i: (i, 0)),
        pl.BlockSpec((1, 64), lambda i: (0, 0)),
    ],
    out_shape=[
        jax.ShapeDtypeStruct((NP, 64), jnp.float32),
        jax.ShapeDtypeStruct((1, 64), jnp.float32),
    ],
)


@functools.partial(
    pl.kernel,
    out_type=jax.ShapeDtypeStruct((NP, 64), jnp.float32),
    mesh=plsc.VectorSubcoreMesh(core_axis_name="c", subcore_axis_name="s"),
    compiler_params=pltpu.CompilerParams(use_tc_tiling_on_sc=False),
    scratch_types=[
        pltpu.VMEM((16, BLK), jnp.int32),
        pltpu.VMEM((BLK, 64), jnp.float32),
        pltpu.VMEM((BLK, 64), jnp.float32),
        pltpu.VMEM((BLK, 64), jnp.float32),
        pltpu.VMEM((64,), jnp.float32),
        pltpu.VMEM((64,), jnp.float32),
        pltpu.VMEM((64,), jnp.float32),
        pltpu.SemaphoreType.DMA,
    ],
)
def _sc_gather(k_hbm, nbr_hbm, th1_hbm, th3_hbm, t2_hbm, out_hbm,
               idx_v, s_v, kl_v, o_v, th1_v, th3_v, t2_v, sem):
    wid = lax.axis_index("s") * NC + lax.axis_index("c")
    pltpu.sync_copy(th1_hbm, th1_v)
    pltpu.sync_copy(th3_hbm, th3_v)
    pltpu.sync_copy(t2_hbm, t2_v)

    zero16 = jnp.zeros((16,), jnp.float32)

    def zero_body(r, carry):
        for c in range(4):
            s_v[r, pl.ds(c * 16, 16)] = zero16
        return carry

    lax.fori_loop(0, BLK, zero_body, 0)

    th1c = [th1_v[pl.ds(c * 16, 16)] for c in range(4)]
    th3c = [th3_v[pl.ds(c * 16, 16)] for c in range(4)]
    t2c = [t2_v[pl.ds(c * 16, 16)] for c in range(4)]

    def blk_body(j, carry):
        b = wid * NBLK + j
        gbase = b * BLK
        pltpu.sync_copy(nbr_hbm.at[b], idx_v)
        cps = [
            pltpu.async_copy(k_hbm.at[idx_v.at[k]], s_v, sem, add=True)
            for k in range(NBR)
        ]
        pltpu.sync_copy(k_hbm.at[pl.ds(gbase, BLK)], kl_v)
        for cp in cps:
            cp.wait()

        def row_body(r, c2):
            for c in range(4):
                sl = pl.ds(c * 16, 16)
                s = s_v[r, sl]
                kk = kl_v[r, sl]
                o_v[r, sl] = th1c[c] * kk + th3c[c] * s + t2c[c]
                s_v[r, sl] = zero16
            return c2

        lax.fori_loop(0, BLK, row_body, 0)
        pltpu.sync_copy(o_v, out_hbm.at[pl.ds(gbase, BLK)])
        return carry

    lax.fori_loop(0, NBLK, blk_body, 0)


def kernel(x, x_v, neighbors_indices, theta1, theta2, theta3):
    n = x.shape[0]
    x2 = x.reshape(n, 2).astype(jnp.float32)
    v32 = x_v.reshape(n, 32).astype(jnp.float32)
    pad = NP - n
    x2 = jnp.concatenate([x2, jnp.zeros((pad, 2), jnp.float32)], axis=0)
    v32 = jnp.concatenate([v32, jnp.zeros((pad, 32), jnp.float32)], axis=0)
    nbr = neighbors_indices.astype(jnp.int32)
    nbr = jnp.concatenate([nbr, jnp.zeros((pad, NBR), jnp.int32)], axis=0)
    nbr_blocked = nbr.reshape(NP // BLK, BLK, NBR).transpose(0, 2, 1)

    k_table, ssum = _kron_call(x2, v32)

    th1v = jnp.tile(theta1.astype(jnp.float32), 8)
    th3v = jnp.tile(theta3.astype(jnp.float32), 8) / NBR
    t2v = jnp.tile(theta2.astype(jnp.float32), 8) * ssum[0] / n

    out = _sc_gather(k_table, nbr_blocked, th1v, th3v, t2v)
    return out[:n].reshape(n, 8, 8)


# speedup vs baseline: 93.4962x; 1.3767x over previous
import functools

import jax
import jax.numpy as jnp
from jax import lax
from jax.experimental import pallas as pl
from jax.experimental.pallas import tpu as pltpu
from jax.experimental.pallas import tpu_sc as plsc

N = 50000
NBR = 16

NC = 2
NS = 16
NW = NC * NS

BLK = 80
NBLKS = N // BLK

RB = 2000


def _kron_body(x_ref, v_ref, k_ref, s_ref):
    i = pl.program_id(0)
    x = x_ref[...]
    v = v_ref[...]
    k = jnp.concatenate([v * x[:, 0:1], v * x[:, 1:2]], axis=1)
    k_ref[...] = k

    @pl.when(i == 0)
    def _():
        s_ref[...] = jnp.zeros_like(s_ref)

    s_ref[...] += jnp.sum(k, axis=0, keepdims=True)


_kron_call = pl.pallas_call(
    _kron_body,
    grid=(N // RB,),
    in_specs=[
        pl.BlockSpec((RB, 2), lambda i: (i, 0)),
        pl.BlockSpec((RB, 32), lambda i: (i, 0)),
    ],
    out_specs=[
        pl.BlockSpec((RB, 64), lambda i: (i, 0)),
        pl.BlockSpec((1, 64), lambda i: (0, 0)),
    ],
    out_shape=[
        jax.ShapeDtypeStruct((N, 64), jnp.float32),
        jax.ShapeDtypeStruct((1, 64), jnp.float32),
    ],
)


@functools.partial(
    pl.kernel,
    out_type=jax.ShapeDtypeStruct((N, 64), jnp.float32),
    mesh=plsc.VectorSubcoreMesh(core_axis_name="c", subcore_axis_name="s"),
    compiler_params=pltpu.CompilerParams(
        use_tc_tiling_on_sc=False, needs_layout_passes=False),
    scratch_types=[
        pltpu.VMEM((BLK, NBR), jnp.int32),
        pltpu.VMEM((2, NBR, BLK), jnp.int32),
        pltpu.VMEM((2, BLK, 64), jnp.float32),
        pltpu.VMEM((BLK, 64), jnp.float32),
        pltpu.VMEM((BLK, 64), jnp.float32),
        pltpu.VMEM((64,), jnp.float32),
        pltpu.VMEM((64,), jnp.float32),
        pltpu.VMEM((64,), jnp.float32),
        pltpu.SemaphoreType.DMA,
        pltpu.SemaphoreType.DMA,
        pltpu.SemaphoreType.DMA,
    ],
)
def _sc_gather(k_hbm, nbr_hbm, th1_hbm, th3_hbm, t2_hbm, out_hbm,
               raw_v, idx_v, s_v, kl_v, o_v, th1_v, th3_v, t2_v,
               sem0, sem1, klsem):
    wid = lax.axis_index("s") * NC + lax.axis_index("c")
    sems = [sem0, sem1]
    pltpu.sync_copy(th1_hbm, th1_v)
    pltpu.sync_copy(th3_hbm, th3_v)
    pltpu.sync_copy(t2_hbm, t2_v)

    zero16 = jnp.zeros((16,), jnp.float32)
    iota = lax.iota(jnp.int32, 16)

    def zero_body(r, carry):
        for p in range(2):
            for c in range(4):
                s_v[p, r, pl.ds(c * 16, 16)] = zero16
        return carry

    lax.fori_loop(0, BLK, zero_body, 0)

    th1c = [th1_v[pl.ds(c * 16, 16)] for c in range(4)]
    th3c = [th3_v[pl.ds(c * 16, 16)] for c in range(4)]
    t2c = [t2_v[pl.ds(c * 16, 16)] for c in range(4)]

    def stage(b, p):
        gbase = b * BLK
        pltpu.sync_copy(nbr_hbm.at[pl.ds(gbase, BLK)], raw_v)
        for k in range(NBR):
            for g in range(BLK // 16):
                rows = g * 16 + iota
                cols = jnp.full((16,), k, jnp.int32)
                vals = plsc.load_gather(raw_v, [rows, cols])
                idx_v[p, k, pl.ds(g * 16, 16)] = vals
        for k in range(NBR):
            pltpu.async_copy(
                k_hbm.at[idx_v.at[p, k]], s_v.at[p],
                sems[p], add=True)

    def finish(b, p):
        gbase = b * BLK
        cp = pltpu.async_copy(k_hbm.at[pl.ds(gbase, BLK)], kl_v, klsem)
        for _ in range(NBR):
            pltpu.make_async_copy(
                k_hbm.at[idx_v.at[p, 0]], s_v.at[p],
                sems[p]).wait()
        cp.wait()

        def row_body(r, carry):
            for c in range(4):
                sl = pl.ds(c * 16, 16)
                s = s_v[p, r, sl]
                kk = kl_v[r, sl]
                o_v[r, sl] = th1c[c] * kk + th3c[c] * s + t2c[c]
                s_v[p, r, sl] = zero16
            return carry

        lax.fori_loop(0, BLK, row_body, 0)
        pltpu.sync_copy(o_v, out_hbm.at[pl.ds(gbase, BLK)])

    nsteps = (NBLKS + NW - 1) // NW
    b0 = wid
    stage(b0, 0)

    def pair_body(jj, carry):
        b_even = wid + (2 * jj) * NW
        b_odd = b_even + NW
        b_next = b_odd + NW

        @pl.when(b_odd < NBLKS)
        def _():
            stage(b_odd, 1)

        finish(b_even, 0)

        @pl.when(b_next < NBLKS)
        def _():
            stage(b_next, 0)

        @pl.when(b_odd < NBLKS)
        def _():
            finish(b_odd, 1)

        return carry

    lax.fori_loop(0, (nsteps + 1) // 2, pair_body, 0)


def kernel(x, x_v, neighbors_indices, theta1, theta2, theta3):
    n = x.shape[0]
    x2 = x.reshape(n, 2).astype(jnp.float32)
    v32 = x_v.reshape(n, 32).astype(jnp.float32)
    nbr = neighbors_indices.astype(jnp.int32)

    k_table, ssum = _kron_call(x2, v32)

    th1v = jnp.tile(theta1.astype(jnp.float32), 8)
    th3v = jnp.tile(theta3.astype(jnp.float32), 8) / NBR
    t2v = jnp.tile(theta2.astype(jnp.float32), 8) * ssum[0] / n

    out = _sc_gather(k_table, nbr, th1v, th3v, t2v)
    return out.reshape(n, 8, 8)


# speedup vs baseline: 101.7145x; 1.0879x over previous
import functools

import jax
import jax.numpy as jnp
from jax import lax
from jax.experimental import pallas as pl
from jax.experimental.pallas import tpu as pltpu
from jax.experimental.pallas import tpu_sc as plsc

N = 50000
NBR = 16

NC = 2
NS = 16
NW = NC * NS

BLK = 80
NBLKS = N // BLK

RB = 2000


def _kron_body(x_ref, v_ref, k_ref, s_ref):
    i = pl.program_id(0)
    x = x_ref[...]
    v = v_ref[...]
    col1 = lax.broadcasted_iota(jnp.int32, (2, 64), 1)
    row1 = lax.broadcasted_iota(jnp.int32, (2, 64), 0)
    e1 = (col1 // 32 == row1).astype(jnp.float32)
    col2 = lax.broadcasted_iota(jnp.int32, (32, 64), 1)
    row2 = lax.broadcasted_iota(jnp.int32, (32, 64), 0)
    e2 = (col2 % 32 == row2).astype(jnp.float32)
    xb = jnp.dot(x, e1, preferred_element_type=jnp.float32,
                 precision=lax.Precision.HIGHEST)
    vb = jnp.dot(v, e2, preferred_element_type=jnp.float32,
                 precision=lax.Precision.HIGHEST)
    k = xb * vb
    k_ref[...] = k

    @pl.when(i == 0)
    def _():
        s_ref[...] = jnp.zeros_like(s_ref)

    s_ref[...] += jnp.sum(k, axis=0, keepdims=True)


_kron_call = pl.pallas_call(
    _kron_body,
    grid=(N // RB,),
    in_specs=[
        pl.BlockSpec((RB, 2), lambda i: (i, 0)),
        pl.BlockSpec((RB, 32), lambda i: (i, 0)),
    ],
    out_specs=[
        pl.BlockSpec((RB, 64), lambda i: (i, 0)),
        pl.BlockSpec((1, 64), lambda i: (0, 0)),
    ],
    out_shape=[
        jax.ShapeDtypeStruct((N, 64), jnp.float32),
        jax.ShapeDtypeStruct((1, 64), jnp.float32),
    ],
)


@functools.partial(
    pl.kernel,
    out_type=jax.ShapeDtypeStruct((N, 64), jnp.float32),
    mesh=plsc.VectorSubcoreMesh(core_axis_name="c", subcore_axis_name="s"),
    compiler_params=pltpu.CompilerParams(
        use_tc_tiling_on_sc=False, needs_layout_passes=False),
    scratch_types=[
        pltpu.VMEM((2, NBR, BLK), jnp.int32),
        pltpu.VMEM((2, BLK, 64), jnp.float32),
        pltpu.VMEM((BLK, 64), jnp.float32),
        pltpu.VMEM((BLK, 64), jnp.float32),
        pltpu.VMEM((64,), jnp.float32),
        pltpu.VMEM((64,), jnp.float32),
        pltpu.VMEM((64,), jnp.float32),
        pltpu.SemaphoreType.DMA,
        pltpu.SemaphoreType.DMA,
        pltpu.SemaphoreType.DMA,
    ],
)
def _sc_gather(k_hbm, nbr_hbm, th1_hbm, th3_hbm, t2_hbm, out_hbm,
               idx_v, s_v, kl_v, o_v, th1_v, th3_v, t2_v,
               sem0, sem1, klsem):
    wid = lax.axis_index("s") * NC + lax.axis_index("c")
    sems = [sem0, sem1]
    pltpu.sync_copy(th1_hbm, th1_v)
    pltpu.sync_copy(th3_hbm, th3_v)
    pltpu.sync_copy(t2_hbm, t2_v)

    zero16 = jnp.zeros((16,), jnp.float32)

    def zero_body(r, carry):
        for p in range(2):
            for c in range(4):
                s_v[p, r, pl.ds(c * 16, 16)] = zero16
        return carry

    lax.fori_loop(0, BLK, zero_body, 0)

    th1c = [th1_v[pl.ds(c * 16, 16)] for c in range(4)]
    th3c = [th3_v[pl.ds(c * 16, 16)] for c in range(4)]
    t2c = [t2_v[pl.ds(c * 16, 16)] for c in range(4)]

    def stage(b, p):
        gbase = b * BLK
        pltpu.sync_copy(nbr_hbm.at[:, pl.ds(gbase, BLK)], idx_v.at[p])
        for k in range(NBR):
            pltpu.async_copy(
                k_hbm.at[idx_v.at[p, k]], s_v.at[p],
                sems[p], add=True)

    def finish(b, p):
        gbase = b * BLK
        cp = pltpu.async_copy(k_hbm.at[pl.ds(gbase, BLK)], kl_v, klsem)
        for _ in range(NBR):
            pltpu.make_async_copy(
                k_hbm.at[idx_v.at[p, 0]], s_v.at[p],
                sems[p]).wait()
        cp.wait()

        def row_body(r, carry):
            for c in range(4):
                sl = pl.ds(c * 16, 16)
                s = s_v[p, r, sl]
                kk = kl_v[r, sl]
                o_v[r, sl] = th1c[c] * kk + th3c[c] * s + t2c[c]
                s_v[p, r, sl] = zero16
            return carry

        lax.fori_loop(0, BLK, row_body, 0)
        pltpu.sync_copy(o_v, out_hbm.at[pl.ds(gbase, BLK)])

    nsteps = (NBLKS + NW - 1) // NW
    b0 = wid
    stage(b0, 0)

    def pair_body(jj, carry):
        b_even = wid + (2 * jj) * NW
        b_odd = b_even + NW
        b_next = b_odd + NW

        @pl.when(b_odd < NBLKS)
        def _():
            stage(b_odd, 1)

        finish(b_even, 0)

        @pl.when(b_next < NBLKS)
        def _():
            stage(b_next, 0)

        @pl.when(b_odd < NBLKS)
        def _():
            finish(b_odd, 1)

        return carry

    lax.fori_loop(0, (nsteps + 1) // 2, pair_body, 0)


def kernel(x, x_v, neighbors_indices, theta1, theta2, theta3):
    n = x.shape[0]
    x2 = x.reshape(n, 2).astype(jnp.float32)
    v32 = x_v.reshape(n, 32).astype(jnp.float32)
    nbrt = neighbors_indices.astype(jnp.int32).T

    k_table, ssum = _kron_call(x2, v32)

    th1v = jnp.tile(theta1.astype(jnp.float32), 8)
    th3v = jnp.tile(theta3.astype(jnp.float32), 8) / NBR
    t2v = jnp.tile(theta2.astype(jnp.float32), 8) * ssum[0] / n

    out = _sc_gather(k_table, nbrt, th1v, th3v, t2v)
    return out.reshape(n, 8, 8)


# speedup vs baseline: 122.2926x; 1.2023x over previous
import functools

import jax
import jax.numpy as jnp
from jax import lax
from jax.experimental import pallas as pl
from jax.experimental.pallas import tpu as pltpu
from jax.experimental.pallas import tpu_sc as plsc

N = 50000
NBR = 16

NC = 2
NS = 16
NW = NC * NS

BLK = 80
NBLKS = N // BLK

RB = 2000


@functools.partial(
    pl.kernel,
    out_type=[
        jax.ShapeDtypeStruct((N, 64), jnp.float32),
        jax.ShapeDtypeStruct((NW, 64), jnp.float32),
    ],
    mesh=plsc.VectorSubcoreMesh(core_axis_name="c", subcore_axis_name="s"),
    compiler_params=pltpu.CompilerParams(
        use_tc_tiling_on_sc=False, needs_layout_passes=False),
    scratch_types=[
        pltpu.VMEM((2, BLK), jnp.float32),
        pltpu.VMEM((32, BLK), jnp.float32),
        pltpu.VMEM((BLK, 64), jnp.float32),
        pltpu.VMEM((64,), jnp.float32),
    ],
)
def _sc_kron(xt_hbm, vt_hbm, k_hbm, t2_hbm, xb_v, vb_v, ko_v, t2s_v):
    wid = lax.axis_index("s") * NC + lax.axis_index("c")
    iota = lax.iota(jnp.int32, 16)
    zero16 = jnp.zeros((16,), jnp.float32)
    nb = jnp.where(wid < (NBLKS % NW), NBLKS // NW + 1, NBLKS // NW)

    def blk_body(j, acc):
        b = wid + j * NW
        gbase = b * BLK
        pltpu.sync_copy(xt_hbm.at[:, pl.ds(gbase, BLK)], xb_v)
        pltpu.sync_copy(vt_hbm.at[:, pl.ds(gbase, BLK)], vb_v)

        def node_body(nn, acc2):
            col = jnp.full((16,), nn, jnp.int32)
            zz = jnp.zeros((16,), jnp.int32)
            x0 = plsc.load_gather(xb_v, [zz, col])
            x1 = plsc.load_gather(xb_v, [zz + 1, col])
            vg0 = plsc.load_gather(vb_v, [iota, col])
            vg1 = plsc.load_gather(vb_v, [16 + iota, col])
            p0, p1, p2, p3 = x0 * vg0, x0 * vg1, x1 * vg0, x1 * vg1
            ko_v[nn, pl.ds(0, 16)] = p0
            ko_v[nn, pl.ds(16, 16)] = p1
            ko_v[nn, pl.ds(32, 16)] = p2
            ko_v[nn, pl.ds(48, 16)] = p3
            a0, a1, a2, a3 = acc2
            return (a0 + p0, a1 + p1, a2 + p2, a3 + p3)

        acc = lax.fori_loop(0, BLK, node_body, acc)
        pltpu.sync_copy(ko_v, k_hbm.at[pl.ds(gbase, BLK)])
        return acc

    acc = lax.fori_loop(0, nb, blk_body, (zero16, zero16, zero16, zero16))
    for c in range(4):
        t2s_v[pl.ds(c * 16, 16)] = acc[c]
    pltpu.sync_copy(t2s_v, t2_hbm.at[wid])


@functools.partial(
    pl.kernel,
    out_type=jax.ShapeDtypeStruct((N, 64), jnp.float32),
    mesh=plsc.VectorSubcoreMesh(core_axis_name="c", subcore_axis_name="s"),
    compiler_params=pltpu.CompilerParams(
        use_tc_tiling_on_sc=False, needs_layout_passes=False),
    scratch_types=[
        pltpu.VMEM((2, NBR, BLK), jnp.int32),
        pltpu.VMEM((2, BLK, 64), jnp.float32),
        pltpu.VMEM((BLK, 64), jnp.float32),
        pltpu.VMEM((BLK, 64), jnp.float32),
        pltpu.VMEM((64,), jnp.float32),
        pltpu.VMEM((64,), jnp.float32),
        pltpu.VMEM((64,), jnp.float32),
        pltpu.SemaphoreType.DMA,
        pltpu.SemaphoreType.DMA,
        pltpu.SemaphoreType.DMA,
    ],
)
def _sc_gather(k_hbm, nbr_hbm, th1_hbm, th3_hbm, t2_hbm, out_hbm,
               idx_v, s_v, kl_v, o_v, th1_v, th3_v, t2_v,
               sem0, sem1, klsem):
    wid = lax.axis_index("s") * NC + lax.axis_index("c")
    sems = [sem0, sem1]
    pltpu.sync_copy(th1_hbm, th1_v)
    pltpu.sync_copy(th3_hbm, th3_v)
    pltpu.sync_copy(t2_hbm, t2_v)

    zero16 = jnp.zeros((16,), jnp.float32)

    def zero_body(r, carry):
        for p in range(2):
            for c in range(4):
                s_v[p, r, pl.ds(c * 16, 16)] = zero16
        return carry

    lax.fori_loop(0, BLK, zero_body, 0)

    th1c = [th1_v[pl.ds(c * 16, 16)] for c in range(4)]
    th3c = [th3_v[pl.ds(c * 16, 16)] for c in range(4)]
    t2c = [t2_v[pl.ds(c * 16, 16)] for c in range(4)]

    def stage(b, p):
        gbase = b * BLK
        pltpu.sync_copy(nbr_hbm.at[:, pl.ds(gbase, BLK)], idx_v.at[p])
        for k in range(NBR):
            pltpu.async_copy(
                k_hbm.at[idx_v.at[p, k]], s_v.at[p],
                sems[p], add=True)

    def finish(b, p):
        gbase = b * BLK
        cp = pltpu.async_copy(k_hbm.at[pl.ds(gbase, BLK)], kl_v, klsem)
        for _ in range(NBR):
            pltpu.make_async_copy(
                k_hbm.at[idx_v.at[p, 0]], s_v.at[p],
                sems[p]).wait()
        cp.wait()

        def row_body(r, carry):
            for c in range(4):
                sl = pl.ds(c * 16, 16)
                s = s_v[p, r, sl]
                kk = kl_v[r, sl]
                o_v[r, sl] = th1c[c] * kk + th3c[c] * s + t2c[c]
                s_v[p, r, sl] = zero16
            return carry

        lax.fori_loop(0, BLK, row_body, 0)
        pltpu.sync_copy(o_v, out_hbm.at[pl.ds(gbase, BLK)])

    nsteps = (NBLKS + NW - 1) // NW
    b0 = wid
    stage(b0, 0)

    def pair_body(jj, carry):
        b_even = wid + (2 * jj) * NW
        b_odd = b_even + NW
        b_next = b_odd + NW

        @pl.when(b_odd < NBLKS)
        def _():
            stage(b_odd, 1)

        finish(b_even, 0)

        @pl.when(b_next < NBLKS)
        def _():
            stage(b_next, 0)

        @pl.when(b_odd < NBLKS)
        def _():
            finish(b_odd, 1)

        return carry

    lax.fori_loop(0, (nsteps + 1) // 2, pair_body, 0)


def kernel(x, x_v, neighbors_indices, theta1, theta2, theta3):
    n = x.shape[0]
    xt = x.reshape(n, 2).astype(jnp.float32).T
    vt = x_v.reshape(n, 32).astype(jnp.float32).T
    nbrt = neighbors_indices.astype(jnp.int32).T

    k_table, t2part = _sc_kron(xt, vt)

    th1v = jnp.tile(theta1.astype(jnp.float32), 8)
    th3v = jnp.tile(theta3.astype(jnp.float32), 8) / NBR
    t2v = jnp.tile(theta2.astype(jnp.float32), 8) * jnp.sum(t2part, axis=0) / n

    out = _sc_gather(k_table, nbrt, th1v, th3v, t2v)
    return out.reshape(n, 8, 8)


# speedup vs baseline: 124.4466x; 1.0176x over previous
import functools

import jax
import jax.numpy as jnp
from jax import lax
from jax.experimental import pallas as pl
from jax.experimental.pallas import tpu as pltpu
from jax.experimental.pallas import tpu_sc as plsc

N = 50000
NBR = 16

NC = 2
NS = 16
NW = NC * NS

BLK = 80
NBLKS = N // BLK

RB = 2000


@functools.partial(
    pl.kernel,
    out_type=[
        jax.ShapeDtypeStruct((N, 64), jnp.float32),
        jax.ShapeDtypeStruct((N, 64), jnp.bfloat16),
        jax.ShapeDtypeStruct((NW, 64), jnp.float32),
    ],
    mesh=plsc.VectorSubcoreMesh(core_axis_name="c", subcore_axis_name="s"),
    compiler_params=pltpu.CompilerParams(
        use_tc_tiling_on_sc=False, needs_layout_passes=False),
    scratch_types=[
        pltpu.VMEM((2, BLK), jnp.float32),
        pltpu.VMEM((32, BLK), jnp.float32),
        pltpu.VMEM((BLK, 64), jnp.float32),
        pltpu.VMEM((BLK, 64), jnp.bfloat16),
        pltpu.VMEM((64,), jnp.float32),
    ],
)
def _sc_kron(xt_hbm, vt_hbm, k_hbm, kb_hbm, t2_hbm,
             xb_v, vb_v, ko_v, kb_v, t2s_v):
    wid = lax.axis_index("s") * NC + lax.axis_index("c")
    iota = lax.iota(jnp.int32, 16)
    zero16 = jnp.zeros((16,), jnp.float32)
    nb = jnp.where(wid < (NBLKS % NW), NBLKS // NW + 1, NBLKS // NW)

    def blk_body(j, acc):
        b = wid + j * NW
        gbase = b * BLK
        pltpu.sync_copy(xt_hbm.at[:, pl.ds(gbase, BLK)], xb_v)
        pltpu.sync_copy(vt_hbm.at[:, pl.ds(gbase, BLK)], vb_v)

        def node_body(nn, acc2):
            col = jnp.full((16,), nn, jnp.int32)
            zz = jnp.zeros((16,), jnp.int32)
            x0 = plsc.load_gather(xb_v, [zz, col])
            x1 = plsc.load_gather(xb_v, [zz + 1, col])
            vg0 = plsc.load_gather(vb_v, [iota, col])
            vg1 = plsc.load_gather(vb_v, [16 + iota, col])
            p0, p1, p2, p3 = x0 * vg0, x0 * vg1, x1 * vg0, x1 * vg1
            ko_v[nn, pl.ds(0, 16)] = p0
            ko_v[nn, pl.ds(16, 16)] = p1
            ko_v[nn, pl.ds(32, 16)] = p2
            ko_v[nn, pl.ds(48, 16)] = p3
            kb_v[nn, pl.ds(0, 32)] = plsc.pack(
                p0, p1, format=plsc.PackFormat.INTERLEAVED)
            kb_v[nn, pl.ds(32, 32)] = plsc.pack(
                p2, p3, format=plsc.PackFormat.INTERLEAVED)
            a0, a1, a2, a3 = acc2
            return (a0 + p0, a1 + p1, a2 + p2, a3 + p3)

        acc = lax.fori_loop(0, BLK, node_body, acc)
        pltpu.sync_copy(ko_v, k_hbm.at[pl.ds(gbase, BLK)])
        pltpu.sync_copy(kb_v, kb_hbm.at[pl.ds(gbase, BLK)])
        return acc

    acc = lax.fori_loop(0, nb, blk_body, (zero16, zero16, zero16, zero16))
    for c in range(4):
        t2s_v[pl.ds(c * 16, 16)] = acc[c]
    pltpu.sync_copy(t2s_v, t2_hbm.at[wid])


@functools.partial(
    pl.kernel,
    out_type=jax.ShapeDtypeStruct((N, 64), jnp.float32),
    mesh=plsc.VectorSubcoreMesh(core_axis_name="c", subcore_axis_name="s"),
    compiler_params=pltpu.CompilerParams(
        use_tc_tiling_on_sc=False, needs_layout_passes=False),
    scratch_types=[
        pltpu.VMEM((2, NBR, BLK), jnp.int32),
        pltpu.VMEM((2, BLK, 64), jnp.bfloat16),
        pltpu.VMEM((BLK, 64), jnp.float32),
        pltpu.VMEM((BLK, 64), jnp.float32),
        pltpu.VMEM((64,), jnp.float32),
        pltpu.VMEM((64,), jnp.float32),
        pltpu.VMEM((64,), jnp.float32),
        pltpu.SemaphoreType.DMA,
        pltpu.SemaphoreType.DMA,
        pltpu.SemaphoreType.DMA,
    ],
)
def _sc_gather(k_hbm, kb_hbm, nbr_hbm, th1_hbm, th3_hbm, t2_hbm, out_hbm,
               idx_v, s_v, kl_v, o_v, th1_v, th3_v, t2_v,
               sem0, sem1, klsem):
    wid = lax.axis_index("s") * NC + lax.axis_index("c")
    sems = [sem0, sem1]
    pltpu.sync_copy(th1_hbm, th1_v)
    pltpu.sync_copy(th3_hbm, th3_v)
    pltpu.sync_copy(t2_hbm, t2_v)

    zero16 = jnp.zeros((16,), jnp.float32)
    zero32b = jnp.zeros((32,), jnp.bfloat16)

    def zero_body(r, carry):
        for p in range(2):
            for c in range(2):
                s_v[p, r, pl.ds(c * 32, 32)] = zero32b
        return carry

    lax.fori_loop(0, BLK, zero_body, 0)

    th1c = [th1_v[pl.ds(c * 16, 16)] for c in range(4)]
    th3c = [th3_v[pl.ds(c * 16, 16)] for c in range(4)]
    t2c = [t2_v[pl.ds(c * 16, 16)] for c in range(4)]

    def stage(b, p):
        gbase = b * BLK
        pltpu.sync_copy(nbr_hbm.at[:, pl.ds(gbase, BLK)], idx_v.at[p])
        for k in range(NBR):
            pltpu.async_copy(
                kb_hbm.at[idx_v.at[p, k]], s_v.at[p],
                sems[p], add=True)

    def finish(b, p):
        gbase = b * BLK
        cp = pltpu.async_copy(k_hbm.at[pl.ds(gbase, BLK)], kl_v, klsem)
        for _ in range(NBR):
            pltpu.make_async_copy(
                kb_hbm.at[idx_v.at[p, 0]], s_v.at[p],
                sems[p]).wait()
        cp.wait()

        himask = jnp.full((16,), -65536, jnp.int32)

        def row_body(r, carry):
            for c in range(2):
                sw = plsc.bitcast(s_v[p, r, pl.ds(c * 32, 32)], jnp.int32)
                s_lo = plsc.bitcast(lax.shift_left(sw, 16), jnp.float32)
                s_hi = plsc.bitcast(sw & himask, jnp.float32)
                for h, s in ((0, s_lo), (1, s_hi)):
                    cc = c * 2 + h
                    sl = pl.ds(cc * 16, 16)
                    kk = kl_v[r, sl]
                    o_v[r, sl] = th1c[cc] * kk + th3c[cc] * s + t2c[cc]
                s_v[p, r, pl.ds(c * 32, 32)] = zero32b
            return carry

        lax.fori_loop(0, BLK, row_body, 0)
        pltpu.sync_copy(o_v, out_hbm.at[pl.ds(gbase, BLK)])

    nsteps = (NBLKS + NW - 1) // NW
    b0 = wid
    stage(b0, 0)

    def pair_body(jj, carry):
        b_even = wid + (2 * jj) * NW
        b_odd = b_even + NW
        b_next = b_odd + NW

        @pl.when(b_odd < NBLKS)
        def _():
            stage(b_odd, 1)

        finish(b_even, 0)

        @pl.when(b_next < NBLKS)
        def _():
            stage(b_next, 0)

        @pl.when(b_odd < NBLKS)
        def _():
            finish(b_odd, 1)

        return carry

    lax.fori_loop(0, (nsteps + 1) // 2, pair_body, 0)


def kernel(x, x_v, neighbors_indices, theta1, theta2, theta3):
    n = x.shape[0]
    xt = x.reshape(n, 2).astype(jnp.float32).T
    vt = x_v.reshape(n, 32).astype(jnp.float32).T
    nbrt = neighbors_indices.astype(jnp.int32).T

    k_table, kb_table, t2part = _sc_kron(xt, vt)

    th1v = jnp.tile(theta1.astype(jnp.float32), 8)
    th3v = jnp.tile(theta3.astype(jnp.float32), 8) / NBR
    t2v = jnp.tile(theta2.astype(jnp.float32), 8) * jnp.sum(t2part, axis=0) / n

    out = _sc_gather(k_table, kb_table, nbrt, th1v, th3v, t2v)
    return out.reshape(n, 8, 8)
